# baseline (speedup 1.0000x reference)
import jax
import jax.numpy as jnp
from jax import lax
from jax.experimental import pallas as pl
from jax.experimental.pallas import tpu as pltpu
from jax.experimental.pallas import tpu_sc as plsc

_BLK = 1024
_L = 256
_SUB = 8


def _sc_body(emb_hbm, pay_hbm, pert_hbm, head_hbm, vemb, vpert, vpay, sem):
    b, s, d = emb_hbm.shape
    wid = lax.axis_index("s") * 2 + lax.axis_index("c")
    bi = wid // 8
    s0 = (wid % 8) * 32

    pltpu.async_copy(pay_hbm, vpay, sem).wait()
    for c in range(32 // _SUB):
        sc0 = s0 + c * _SUB
        pltpu.async_copy(
            emb_hbm.at[pl.ds(bi, 1), pl.ds(sc0, _SUB), :], vemb, sem
        ).wait()
        pltpu.async_copy(pert_hbm.at[pl.ds(sc0, _SUB), :], vpert, sem).wait()
        for i in range(_SUB):
            idx = sc0 + jnp.full((16,), i, jnp.int32)
            pay_i = plsc.load_gather(vpay, [idx])
            maskf = (pay_i == 1).astype(jnp.float32)

            def _chunk(j, _, i=i, maskf=maskf):
                sl = pl.ds(j * 16, 16)
                vemb[0, i, sl] = vemb[0, i, sl] + vpert[i, sl] * maskf
                return 0

            lax.fori_loop(0, d // 16, _chunk, 0)
        pltpu.async_copy(
            vemb, head_hbm.at[pl.ds(bi, 1), pl.ds(sc0, _SUB), :], sem
        ).wait()


def _sc_head(embedded_input, watermark_payload, perturbation_vectors):
    b, s, d = embedded_input.shape
    l = perturbation_vectors.shape[0]
    mesh = plsc.VectorSubcoreMesh(core_axis_name="c", subcore_axis_name="s")
    return pl.kernel(
        _sc_body,
        mesh=mesh,
        out_type=jax.ShapeDtypeStruct((b, l, d), jnp.float32),
        compiler_params=pltpu.CompilerParams(needs_layout_passes=False),
        scratch_types=[
            pltpu.VMEM((1, _SUB, d), jnp.float32),
            pltpu.VMEM((_SUB, d), jnp.float32),
            pltpu.VMEM((l,), jnp.int32),
            pltpu.SemaphoreType.DMA,
        ],
    )(embedded_input, watermark_payload, perturbation_vectors)


def _tc_body(head_ref, emb_ref, out_ref):
    j = pl.program_id(1)

    @pl.when(j == 0)
    def _():
        out_ref[0, :_L, :] = head_ref[0]
        out_ref[0, _L:, :] = emb_ref[0, _L:, :]

    @pl.when(j != 0)
    def _():
        out_ref[...] = emb_ref[...]


def kernel(embedded_input, watermark_payload, perturbation_vectors):
    b, s, d = embedded_input.shape
    l = perturbation_vectors.shape[0]
    head = _sc_head(embedded_input, watermark_payload, perturbation_vectors)
    return pl.pallas_call(
        _tc_body,
        grid=(b, s // _BLK),
        in_specs=[
            pl.BlockSpec((1, l, d), lambda bi, j: (bi, 0, 0)),
            pl.BlockSpec((1, _BLK, d), lambda bi, j: (bi, j, 0)),
        ],
        out_specs=pl.BlockSpec((1, _BLK, d), lambda bi, j: (bi, j, 0)),
        out_shape=jax.ShapeDtypeStruct((b, s, d), embedded_input.dtype),
    )(head, embedded_input)


# speedup vs baseline: 1.6684x; 1.6684x over previous
import jax
import jax.numpy as jnp
from jax.experimental import pallas as pl
from jax.experimental.pallas import tpu as pltpu

_BLK = 1024
_L = 256


def _body(pay_ref, pert_ref, emb_ref, out_ref):
    j = pl.program_id(1)

    @pl.when(j == 0)
    def _():
        mask = (pay_ref[...] == 1).astype(out_ref.dtype)
        out_ref[0, :_L, :] = emb_ref[0, :_L, :] + pert_ref[...] * mask
        out_ref[0, _L:, :] = emb_ref[0, _L:, :]

    @pl.when(j != 0)
    def _():
        out_ref[...] = emb_ref[...]


def kernel(embedded_input, watermark_payload, perturbation_vectors):
    b, s, d = embedded_input.shape
    l = perturbation_vectors.shape[0]
    pay2d = watermark_payload.reshape(l, 1)
    return pl.pallas_call(
        _body,
        grid=(b, s // _BLK),
        in_specs=[
            pl.BlockSpec((l, 1), lambda bi, j: (0, 0)),
            pl.BlockSpec((l, d), lambda bi, j: (0, 0)),
            pl.BlockSpec((1, _BLK, d), lambda bi, j: (bi, j, 0)),
        ],
        out_specs=pl.BlockSpec((1, _BLK, d), lambda bi, j: (bi, j, 0)),
        out_shape=jax.ShapeDtypeStruct((b, s, d), embedded_input.dtype),
    )(pay2d, perturbation_vectors, embedded_input)


# speedup vs baseline: 1.6706x; 1.0013x over previous
import jax
import jax.numpy as jnp
from jax.experimental import pallas as pl
from jax.experimental.pallas import tpu as pltpu

_BLK = 1024
_L = 256


def _body(pay_ref, pert_ref, emb_ref, out_ref):
    j = pl.program_id(1)

    @pl.when(j == 0)
    def _():
        mask = (pay_ref[...] == 1).astype(out_ref.dtype)
        out_ref[0, :_L, :] = emb_ref[0, :_L, :] + pert_ref[...] * mask
        out_ref[0, _L:, :] = emb_ref[0, _L:, :]

    @pl.when(j != 0)
    def _():
        out_ref[...] = emb_ref[...]


def kernel(embedded_input, watermark_payload, perturbation_vectors):
    b, s, d = embedded_input.shape
    l = perturbation_vectors.shape[0]
    pay2d = watermark_payload.reshape(l, 1)
    return pl.pallas_call(
        _body,
        grid=(b, s // _BLK),
        in_specs=[
            pl.BlockSpec((l, 1), lambda bi, j: (0, 0)),
            pl.BlockSpec((l, d), lambda bi, j: (0, 0)),
            pl.BlockSpec((1, _BLK, d), lambda bi, j: (bi, j, 0)),
        ],
        out_specs=pl.BlockSpec((1, _BLK, d), lambda bi, j: (bi, j, 0)),
        out_shape=jax.ShapeDtypeStruct((b, s, d), embedded_input.dtype),
        compiler_params=pltpu.CompilerParams(
            dimension_semantics=("parallel", "arbitrary"),
        ),
    )(pay2d, perturbation_vectors, embedded_input)
